# baseline (speedup 1.0000x reference)
import functools

import jax
import jax.numpy as jnp
from jax import lax
from jax.experimental import pallas as pl
from jax.experimental.pallas import tpu as pltpu
from jax.experimental.pallas import tpu_sc as plsc

B, L, D = 4096, 200, 64
N = B * L
NC, NS = 2, 16
NW = NC * NS
PER_W = N // NW
CHUNK = 400
NCH = PER_W // CHUNK
GSTREAMS = 4
GLEN = CHUNK // GSTREAMS
SEQ_MAJ = N // GLEN


def _make_kernel():
  mesh = plsc.VectorSubcoreMesh(core_axis_name="c", subcore_axis_name="s")

  @functools.partial(
      pl.kernel,
      mesh=mesh,
      compiler_params=pltpu.CompilerParams(use_tc_tiling_on_sc=False),
      out_type=jax.ShapeDtypeStruct((N, D), jnp.float32),
      scratch_types=[
          pltpu.VMEM((PER_W // GLEN, GLEN), jnp.int32),
          pltpu.VMEM((CHUNK, D), jnp.float32),
          pltpu.VMEM((CHUNK, D), jnp.float32),
          pltpu.VMEM((CHUNK, D), jnp.float32),
          pltpu.SemaphoreType.DMA,
          pltpu.SemaphoreType.DMA,
      ],
  )
  def emb_kernel(seq_hbm, table_hbm, pe_hbm, out_hbm,
                 idx_all, rows0, rows1, pe_v, sem0, sem1):
    wid = lax.axis_index("s") * NC + lax.axis_index("c")
    row_base = wid * PER_W

    idx_off = pl.multiple_of(row_base // GLEN, 8)
    pltpu.sync_copy(seq_hbm.at[pl.ds(idx_off, PER_W // GLEN)], idx_all)
    pltpu.sync_copy(pe_hbm.at[pl.ds(0, L)], pe_v.at[pl.ds(0, L)])
    pltpu.sync_copy(pe_hbm.at[pl.ds(0, L)], pe_v.at[pl.ds(L, L)])

    def fire(g, rows_v, sem):
      for j in range(GSTREAMS):
        pltpu.async_copy(table_hbm.at[idx_all.at[g * GSTREAMS + j]],
                         rows_v.at[pl.ds(j * GLEN, GLEN)], sem)

    def drain(rows_v, sem):
      pltpu.make_async_copy(table_hbm.at[pl.ds(0, CHUNK)], rows_v, sem).wait()

    def add_pe(rows_v):
      def body(r, carry):
        for d in range(D // 16):
          sl = pl.ds(d * 16, 16)
          rows_v[r, sl] = rows_v[r, sl] + pe_v[r, sl]
        return carry
      lax.fori_loop(0, CHUNK, body, 0)

    def write(g, rows_v):
      pltpu.sync_copy(rows_v, out_hbm.at[pl.ds(row_base + g * CHUNK, CHUNK)])

    bufs = ((rows0, sem0), (rows1, sem1))
    fire(0, *bufs[0])
    fire(1, *bufs[1])

    def loop_body(k, carry):
      g0 = k * 2
      for b in range(2):
        rv, sm = bufs[b]
        drain(rv, sm)
        add_pe(rv)
        write(g0 + b, rv)
        fire(g0 + b + 2, rv, sm)
      return carry

    lax.fori_loop(0, (NCH - 2) // 2, loop_body, 0)
    for b in range(2):
      rv, sm = bufs[b]
      drain(rv, sm)
      add_pe(rv)
      write(NCH - 2 + b, rv)

  return emb_kernel


_emb_kernel = _make_kernel()


@jax.jit
def kernel(sequence, token_table, pe):
  seq2d = sequence.astype(jnp.int32).reshape(SEQ_MAJ, GLEN)
  out = _emb_kernel(seq2d, token_table, pe)
  return out.reshape(B, L, D)
